# baseline (speedup 1.0000x reference)
import functools

import jax
import jax.numpy as jnp
from jax import lax
from jax.experimental import pallas as pl
from jax.experimental.pallas import tpu as pltpu

N_BINS = 15
N_ROWS = 16384
N_COLS = 1000
BLOCK_ROWS = 256


def _ece_kernel(logits_ref, labels_ref, lo_ref, hi_ref, out_ref,
                cnt_acc, conf_acc, acc_acc, *, nsteps, n_total):
    step = pl.program_id(0)

    x = logits_ref[...]
    m = jnp.max(x, axis=1, keepdims=True)
    s = jnp.sum(jnp.exp(x - m), axis=1)
    conf = 1.0 / s

    col = lax.broadcasted_iota(jnp.int32, x.shape, 1)
    idx = jnp.min(jnp.where(x == m, col, N_COLS), axis=1)
    lab = labels_ref[0, 0, :]
    acc = (idx == lab).astype(jnp.float32)

    confs = conf[None, :]
    mask = jnp.logical_and(confs > lo_ref[...], confs <= hi_ref[...])
    mask = mask.astype(jnp.float32)

    @pl.when(step == 0)
    def _init():
        cnt_acc[...] = jnp.zeros_like(cnt_acc)
        conf_acc[...] = jnp.zeros_like(conf_acc)
        acc_acc[...] = jnp.zeros_like(acc_acc)

    cnt_acc[...] += mask
    conf_acc[...] += mask * confs
    acc_acc[...] += mask * acc[None, :]

    @pl.when(step == nsteps - 1)
    def _finish():
        cnt = jnp.sum(cnt_acc[...], axis=1)
        csum = jnp.sum(conf_acc[...], axis=1)
        asum = jnp.sum(acc_acc[...], axis=1)
        prop = cnt / n_total
        denom = jnp.maximum(cnt, 1.0)
        contrib = jnp.abs(csum / denom - asum / denom) * prop
        out_ref[0, 0] = jnp.sum(jnp.where(cnt > 0.0, contrib, 0.0))


@jax.jit
def kernel(logits, labels):
    nsteps = N_ROWS // BLOCK_ROWS
    labels3d = labels.reshape(nsteps, 1, BLOCK_ROWS)

    bounds = jnp.linspace(0.0, 1.0, N_BINS + 1).astype(jnp.float32)
    lo = jnp.concatenate([bounds[:N_BINS], jnp.ones((1,), jnp.float32)])
    hi = jnp.concatenate([bounds[1:], jnp.ones((1,), jnp.float32)])
    lo2d = jnp.broadcast_to(lo[:, None], (16, BLOCK_ROWS))
    hi2d = jnp.broadcast_to(hi[:, None], (16, BLOCK_ROWS))

    out = pl.pallas_call(
        functools.partial(_ece_kernel, nsteps=nsteps, n_total=float(N_ROWS)),
        grid=(nsteps,),
        in_specs=[
            pl.BlockSpec((BLOCK_ROWS, N_COLS), lambda i: (i, 0)),
            pl.BlockSpec((1, 1, BLOCK_ROWS), lambda i: (i, 0, 0)),
            pl.BlockSpec((16, BLOCK_ROWS), lambda i: (0, 0)),
            pl.BlockSpec((16, BLOCK_ROWS), lambda i: (0, 0)),
        ],
        out_specs=pl.BlockSpec(
            (1, 1), lambda i: (0, 0), memory_space=pltpu.SMEM),
        out_shape=jax.ShapeDtypeStruct((1, 1), jnp.float32),
        scratch_shapes=[
            pltpu.VMEM((16, BLOCK_ROWS), jnp.float32),
            pltpu.VMEM((16, BLOCK_ROWS), jnp.float32),
            pltpu.VMEM((16, BLOCK_ROWS), jnp.float32),
        ],
        compiler_params=pltpu.CompilerParams(
            dimension_semantics=("arbitrary",)),
    )(logits, labels3d, lo2d, hi2d)
    return out[0, 0]


# speedup vs baseline: 1.9398x; 1.9398x over previous
import functools

import jax
import jax.numpy as jnp
from jax import lax
from jax.experimental import pallas as pl
from jax.experimental.pallas import tpu as pltpu

N_BINS = 15
N_ROWS = 16384
N_COLS = 1000
BLOCK_ROWS = 256


def _ece_kernel(logits_ref, labels_ref, lo_ref, hi_ref, out_ref,
                cnt_acc, conf_acc, acc_acc, *, nsteps, n_total):
    step = pl.program_id(0)

    x = logits_ref[...]
    m = jnp.max(x, axis=1, keepdims=True)
    s = jnp.sum(jnp.exp(x - m), axis=1, keepdims=True)
    conf = 1.0 / s

    col = lax.broadcasted_iota(jnp.int32, x.shape, 1)
    idx = jnp.min(jnp.where(x == m, col, N_COLS), axis=1,
                  keepdims=True)
    acc = (idx == labels_ref[0]).astype(jnp.float32)

    mask = jnp.logical_and(conf > lo_ref[...], conf <= hi_ref[...])
    mask = mask.astype(jnp.float32)

    @pl.when(step == 0)
    def _init():
        cnt_acc[...] = jnp.zeros_like(cnt_acc)
        conf_acc[...] = jnp.zeros_like(conf_acc)
        acc_acc[...] = jnp.zeros_like(acc_acc)

    cnt_acc[...] += mask
    conf_acc[...] += mask * conf
    acc_acc[...] += mask * acc

    @pl.when(step == nsteps - 1)
    def _finish():
        cnt = jnp.sum(cnt_acc[...], axis=0)
        csum = jnp.sum(conf_acc[...], axis=0)
        asum = jnp.sum(acc_acc[...], axis=0)
        prop = cnt / n_total
        denom = jnp.maximum(cnt, 1.0)
        contrib = jnp.abs(csum / denom - asum / denom) * prop
        out_ref[0, 0] = jnp.sum(jnp.where(cnt > 0.0, contrib, 0.0))


@jax.jit
def kernel(logits, labels):
    nsteps = N_ROWS // BLOCK_ROWS
    labels3d = labels.reshape(nsteps, BLOCK_ROWS, 1)

    bounds = jnp.linspace(0.0, 1.0, N_BINS + 1).astype(jnp.float32)
    lo = jnp.concatenate([bounds[:N_BINS], jnp.ones((1,), jnp.float32)])
    hi = jnp.concatenate([bounds[1:], jnp.ones((1,), jnp.float32)])
    lo2d = lo.reshape(1, 16)
    hi2d = hi.reshape(1, 16)

    out = pl.pallas_call(
        functools.partial(_ece_kernel, nsteps=nsteps, n_total=float(N_ROWS)),
        grid=(nsteps,),
        in_specs=[
            pl.BlockSpec((BLOCK_ROWS, N_COLS), lambda i: (i, 0)),
            pl.BlockSpec((1, BLOCK_ROWS, 1), lambda i: (i, 0, 0)),
            pl.BlockSpec((1, 16), lambda i: (0, 0)),
            pl.BlockSpec((1, 16), lambda i: (0, 0)),
        ],
        out_specs=pl.BlockSpec(
            (1, 1), lambda i: (0, 0), memory_space=pltpu.SMEM),
        out_shape=jax.ShapeDtypeStruct((1, 1), jnp.float32),
        scratch_shapes=[
            pltpu.VMEM((BLOCK_ROWS, 16), jnp.float32),
            pltpu.VMEM((BLOCK_ROWS, 16), jnp.float32),
            pltpu.VMEM((BLOCK_ROWS, 16), jnp.float32),
        ],
        compiler_params=pltpu.CompilerParams(
            dimension_semantics=("arbitrary",)),
    )(logits, labels3d, lo2d, hi2d)
    return out[0, 0]
